# Initial kernel scaffold; baseline (speedup 1.0000x reference)
import jax
import jax.numpy as jnp
from jax.experimental import pallas as pl


def kernel(x, edge_index, W1, b1, W2, b2, Wc, bc):
    raise NotImplementedError("write your pallas kernel here")


# speedup vs baseline: 6.3789x; 6.3789x over previous
import functools

import jax
import jax.numpy as jnp
from jax import lax
from jax.experimental import pallas as pl
from jax.experimental.pallas import tpu as pltpu
from jax.experimental.pallas import tpu_sc as plsc

N = 50000
E = 800000
NPAD = 50176
EPAD = 802816
NC, NS = 2, 16
K = 128
BIN = NPAD // 4
ACC128 = BIN + 128
CH32 = EPAD // (NC * NS * K)
CH16 = EPAD // (NS * K)
RB = 512
GRID = NPAD // RB

_mesh = plsc.VectorSubcoreMesh(core_axis_name="c", subcore_axis_name="s")
_sc_params = pltpu.CompilerParams(use_tc_tiling_on_sc=False)


@functools.partial(
    pl.kernel,
    out_type=jax.ShapeDtypeStruct((NC, NPAD, 16), jnp.float32),
    mesh=_mesh,
    compiler_params=_sc_params,
    scratch_types=[
        pltpu.VMEM((1, K), jnp.int32),
        pltpu.VMEM((K, 16), jnp.float32),
        pltpu.VMEM_SHARED((NPAD, 16), jnp.float32),
    ],
)
def _sc_deg(dst_hbm, ones_hbm, zeros_hbm, out_hbm, dbuf, ones_v, acc):
    c = lax.axis_index("c")
    s = lax.axis_index("s")
    rows = NPAD // NS
    pltpu.sync_copy(zeros_hbm, acc.at[pl.ds(s * rows, rows)])
    pltpu.sync_copy(ones_hbm, ones_v)
    plsc.subcore_barrier()
    wid = s * NC + c
    base = wid * (EPAD // (NC * NS))

    @pl.loop(0, CH32)
    def _(i):
        pltpu.sync_copy(dst_hbm.at[pl.ds(base + i * K, K)], dbuf.at[0])
        pltpu.sync_copy(ones_v, acc.at[dbuf.at[0]], add=True)

    plsc.subcore_barrier()
    pltpu.sync_copy(acc.at[pl.ds(s * rows, rows)],
                    out_hbm.at[c, pl.ds(s * rows, rows)])


@functools.partial(
    pl.kernel,
    out_type=jax.ShapeDtypeStruct((NC, NPAD, 16), jnp.float32),
    mesh=_mesh,
    compiler_params=_sc_params,
    scratch_types=[
        pltpu.VMEM((1, K), jnp.int32),
        pltpu.VMEM((1, K), jnp.int32),
        pltpu.VMEM((K, 16), jnp.float32),
        pltpu.VMEM_SHARED((NPAD, 16), jnp.float32),
        pltpu.SemaphoreType.DMA,
    ],
)
def _sc_agg16(t_hbm, src_hbm, dst_hbm, zeros_hbm, out_hbm,
              sbuf, dbuf, rows_v, acc, sem):
    c = lax.axis_index("c")
    s = lax.axis_index("s")
    rows = NPAD // NS
    pltpu.sync_copy(zeros_hbm, acc.at[pl.ds(s * rows, rows)])
    plsc.subcore_barrier()
    wid = s * NC + c
    base = wid * (EPAD // (NC * NS))

    @pl.loop(0, CH32)
    def _(i):
        off = base + i * K
        pltpu.sync_copy(src_hbm.at[pl.ds(off, K)], sbuf.at[0])
        cp = pltpu.async_copy(t_hbm.at[sbuf.at[0]], rows_v, sem)
        pltpu.sync_copy(dst_hbm.at[pl.ds(off, K)], dbuf.at[0])
        cp.wait()
        pltpu.sync_copy(rows_v, acc.at[dbuf.at[0]], add=True)

    plsc.subcore_barrier()
    pltpu.sync_copy(acc.at[pl.ds(s * rows, rows)],
                    out_hbm.at[c, pl.ds(s * rows, rows)])


@functools.partial(
    pl.kernel,
    out_type=jax.ShapeDtypeStruct((4, BIN, 128), jnp.float32),
    mesh=_mesh,
    compiler_params=_sc_params,
    scratch_types=[
        pltpu.VMEM((1, K), jnp.int32),
        pltpu.VMEM((1, K), jnp.int32),
        pltpu.VMEM((1, K), jnp.int32),
        pltpu.VMEM((K, 128), jnp.float32),
        pltpu.VMEM_SHARED((ACC128, 128), jnp.float32),
        pltpu.SemaphoreType.DMA,
    ],
)
def _sc_agg128(t_hbm, src_hbm, dst_hbm, zeros_hbm, out_hbm,
               sbuf, dbuf, lbuf, rows_v, acc, sem):
    c = lax.axis_index("c")
    s = lax.axis_index("s")
    zrows = ACC128 // NS
    orows = BIN // NS
    base = s * (EPAD // NS)

    for bi in range(2):
        b = 2 * c + bi
        lo = b * BIN
        pltpu.sync_copy(zeros_hbm, acc.at[pl.ds(s * zrows, zrows)])
        plsc.subcore_barrier()

        @pl.loop(0, CH16)
        def _(i):
            off = base + i * K
            pltpu.sync_copy(src_hbm.at[pl.ds(off, K)], sbuf.at[0])
            cp = pltpu.async_copy(t_hbm.at[sbuf.at[0]], rows_v, sem)
            pltpu.sync_copy(dst_hbm.at[pl.ds(off, K)], dbuf.at[0])
            for g in range(K // 16):
                dv = dbuf[0, pl.ds(g * 16, 16)]
                inb = (dv >= lo) & (dv < lo + BIN)
                loc = jnp.where(inb, dv - lo, BIN + (dv & 127))
                lbuf[0, pl.ds(g * 16, 16)] = loc
            cp.wait()
            pltpu.sync_copy(rows_v, acc.at[lbuf.at[0]], add=True)

        plsc.subcore_barrier()
        pltpu.sync_copy(acc.at[pl.ds(s * orows, orows)],
                        out_hbm.at[b, pl.ds(s * orows, orows)])
        plsc.subcore_barrier()


def _tc_call(body, out_specs, out_types, *args_specs):
    def wrap(*arrays):
        return pl.pallas_call(
            body,
            grid=(GRID,),
            in_specs=list(args_specs),
            out_specs=out_specs,
            out_shape=out_types,
        )(*arrays)
    return wrap


_row_spec16 = pl.BlockSpec((RB, 16), lambda i: (i, 0))
_row_spec128 = pl.BlockSpec((RB, 128), lambda i: (i, 0))
_part_spec = pl.BlockSpec((NC, RB, 16), lambda i: (0, i, 0))
_full128 = pl.BlockSpec((128, 128), lambda i: (0, 0))
_brow128 = pl.BlockSpec((1, 128), lambda i: (0, 0))


def _deg_body(p_ref, x_ref, dinv_ref, t1_ref):
    cnt = p_ref[0] + p_ref[1]
    dinv = lax.rsqrt(cnt + 1.0)
    dinv_ref[...] = dinv
    x = x_ref[...]
    t1 = dinv[:, :4] * x
    t1_ref[...] = jnp.concatenate(
        [t1, jnp.zeros((RB, 12), jnp.float32)], axis=1)


def _layer1_body(p_ref, x_ref, dinv_ref, w1_ref, b1_ref, t2_ref):
    i = pl.program_id(0)
    g = p_ref[0, :, :4] + p_ref[1, :, :4]
    dv4 = dinv_ref[:, :4]
    pre = dv4 * g + dv4 * dv4 * x_ref[...]
    h = jnp.maximum(
        jnp.dot(pre, w1_ref[...], preferred_element_type=jnp.float32)
        + b1_ref[...], 0.0)
    t2 = dinv_ref[:, 0:1] * h
    row = i * RB + lax.broadcasted_iota(jnp.int32, (RB, 1), 0)
    t2_ref[...] = jnp.where(row < N, t2, 0.0)


def _layer_body(g_ref, t_ref, dinv_ref, w2_ref, b2_ref, o_ref):
    i = pl.program_id(0)
    dvb = dinv_ref[:, 0:1]
    pre = dvb * (g_ref[...] + t_ref[...])
    h = jnp.maximum(
        jnp.dot(pre, w2_ref[...], preferred_element_type=jnp.float32)
        + b2_ref[...], 0.0)
    t = dvb * h
    row = i * RB + lax.broadcasted_iota(jnp.int32, (RB, 1), 0)
    o_ref[...] = jnp.where(row < N, t, 0.0)


def _final_body(g_ref, t_ref, dinv_ref, w2_ref, b2_ref, wc_ref, bc_ref, o_ref):
    dvb = dinv_ref[:, 0:1]
    pre = dvb * (g_ref[...] + t_ref[...])
    h = jnp.maximum(
        jnp.dot(pre, w2_ref[...], preferred_element_type=jnp.float32)
        + b2_ref[...], 0.0)
    o_ref[...] = (jnp.dot(h, wc_ref[...], preferred_element_type=jnp.float32)
                  + bc_ref[...])


_f32 = jnp.float32


def kernel(x, edge_index, W1, b1, W2, b2, Wc, bc):
    npad_x = jnp.zeros((NPAD, 4), _f32).at[:N].set(x)
    pad = jnp.arange(EPAD - E, dtype=jnp.int32)
    src = jnp.concatenate([edge_index[0], N + (pad % 8)])
    dst = jnp.concatenate([edge_index[1], N + (pad % 128)])
    ones16 = jnp.ones((K, 16), _f32)
    z16 = jnp.zeros((NPAD // NS, 16), _f32)
    z128 = jnp.zeros((ACC128 // NS, 128), _f32)
    b1r = b1.reshape(1, 128)
    b2r = b2.reshape(1, 128)
    wc_pad = jnp.zeros((128, 128), _f32).at[:, :2].set(Wc)
    bc_pad = jnp.zeros((1, 128), _f32).at[:, :2].set(bc)

    deg_parts = _sc_deg(dst, ones16, z16)

    dinv16, t1 = _tc_call(
        _deg_body,
        [_row_spec16, _row_spec16],
        [jax.ShapeDtypeStruct((NPAD, 16), _f32),
         jax.ShapeDtypeStruct((NPAD, 16), _f32)],
        _part_spec, pl.BlockSpec((RB, 4), lambda i: (i, 0)),
    )(deg_parts, npad_x)

    agg16_parts = _sc_agg16(t1, src, dst, z16)

    (t2,) = _tc_call(
        _layer1_body,
        [_row_spec128],
        [jax.ShapeDtypeStruct((NPAD, 128), _f32)],
        _part_spec, pl.BlockSpec((RB, 4), lambda i: (i, 0)),
        _row_spec16, pl.BlockSpec((4, 128), lambda i: (0, 0)), _brow128,
    )(agg16_parts, npad_x, dinv16, W1, b1r)

    g2 = _sc_agg128(t2, src, dst, z128).reshape(NPAD, 128)
    (t3,) = _tc_call(
        _layer_body,
        [_row_spec128],
        [jax.ShapeDtypeStruct((NPAD, 128), _f32)],
        _row_spec128, _row_spec128, _row_spec16, _full128, _brow128,
    )(g2, t2, dinv16, W2, b2r)

    g3 = _sc_agg128(t3, src, dst, z128).reshape(NPAD, 128)
    out = _tc_call(
        _final_body,
        _row_spec128,
        jax.ShapeDtypeStruct((NPAD, 128), _f32),
        _row_spec128, _row_spec128, _row_spec16, _full128, _brow128,
        _full128, _brow128,
    )(g3, t3, dinv16, W2, b2r, wc_pad, bc_pad)

    return out[:N, :2]
